# baseline (speedup 1.0000x reference)
import jax
import jax.numpy as jnp
from jax import lax
from jax.experimental import pallas as pl
from jax.experimental.pallas import tpu as pltpu
from jax.experimental.pallas import tpu_sc as plsc

B = 4096
L = 200
D = 64
EPS = 1e-5

NC = 2
NS = 16
NW = NC * NS

ROWS = 400
NIT = (B * L) // (NW * ROWS)
GCH = 100
NGC = ROWS // GCH
OUTF = ROWS * D

MAGIC = 0x5F3759DF


def _rsqrt(v):
    i = plsc.bitcast(v, jnp.int32)
    i = MAGIC - lax.shift_right_logical(i, 1)
    y = plsc.bitcast(i, jnp.float32)
    hv = 0.5 * v
    y = y * (1.5 - hv * y * y)
    y = y * (1.5 - hv * y * y)
    y = y * (1.5 - hv * y * y)
    return y


def _sc_kernel(ids_hbm, table_hbm, pos_hbm, gam_hbm, bet_hbm, out_hbm,
               idxA, idxB, inA, inB, outA, outB, pos_v, gam_v, bet_v,
               gsemA, gsemB, osemA, osemB, isemA, isemB):
    cid = lax.axis_index("c")
    sid = lax.axis_index("s")
    wid = cid * NS + sid

    idx = [idxA, idxB]
    inb = [inA, inB]
    outb = [outA, outB]
    gsem = [gsemA, gsemB]
    osem = [osemA, osemB]
    isem = [isemA, isemB]

    pltpu.sync_copy(pos_hbm.at[pl.ds(0, L * D)], pos_v)
    pltpu.sync_copy(gam_hbm, gam_v)
    pltpu.sync_copy(bet_hbm, bet_v)

    g = [gam_v[pl.ds(16 * k, 16)] for k in range(4)]
    bt = [bet_v[pl.ds(16 * k, 16)] for k in range(4)]

    def fire_ids(t, q):
        return pltpu.async_copy(
            ids_hbm.at[pl.ds((wid * NIT + t) * NGC, NGC), :], idx[q], isem[q])

    def wait_ids(q):
        pltpu.make_async_copy(ids_hbm.at[pl.ds(0, NGC), :], idx[q],
                              isem[q]).wait()

    def fire_gather(q):
        for j in range(NGC):
            pltpu.async_copy(table_hbm.at[idx[q].at[j]],
                             inb[q].at[pl.ds(j * GCH, GCH), :], gsem[q])

    def wait_gather(q):
        pltpu.make_async_copy(table_hbm.at[pl.ds(0, ROWS), :], inb[q],
                              gsem[q]).wait()

    def fire_out(t, q):
        return pltpu.async_copy(
            outb[q], out_hbm.at[pl.ds((wid * NIT + t) * OUTF, OUTF)], osem[q])

    def wait_out(q):
        pltpu.make_async_copy(out_hbm.at[pl.ds(0, OUTF)], outb[q],
                              osem[q]).wait()

    fire_ids(0, 0)
    wait_ids(0)
    fire_ids(1, 1)
    fire_gather(0)

    @pl.loop(0, NIT, step=2)
    def _iter2(t0):
        for p in (0, 1):
            t = t0 + p
            q = 1 - p

            @pl.when(t < NIT - 1)
            def _prefetch():
                wait_ids(q)
                fire_gather(q)

            wait_gather(p)

            @pl.when(t < NIT - 2)
            def _nextids():
                fire_ids(t + 2, p)

            @pl.when(t >= 2)
            def _drainout():
                wait_out(p)

            src = inb[p]
            dst = outb[p]

            @pl.loop(0, ROWS)
            def _row(r):
                l = lax.rem(r, L)
                po = l * D
                x = [src[r, pl.ds(16 * k, 16)] + pos_v[pl.ds(po + 16 * k, 16)]
                     for k in range(4)]
                tot = (x[0] + x[1]) + (x[2] + x[3])
                qq = ((x[0] * x[0] + x[1] * x[1])
                      + (x[2] * x[2] + x[3] * x[3]))
                sv = jnp.full((16,), jnp.sum(tot))
                qv = jnp.full((16,), jnp.sum(qq))
                mean = sv * (1.0 / D)
                var = qv * (1.0 / D) - mean * mean
                rstd = _rsqrt(var + EPS)
                rg = [rstd * g[k] for k in range(4)]
                base = r * D
                for k in range(4):
                    y = (x[k] - mean) * rg[k] + bt[k]
                    dst[pl.ds(base + 16 * k, 16)] = y

            fire_out(t, p)

    wait_out(0)
    wait_out(1)


@jax.jit
def kernel(input_ids_BL, gene_table, pos_table, ln_gamma, ln_beta):
    ids2 = input_ids_BL.astype(jnp.int32).reshape(-1, GCH)
    pos_flat = pos_table.reshape(-1)

    mesh = plsc.VectorSubcoreMesh(core_axis_name="c", subcore_axis_name="s",
                                  num_cores=NC, num_subcores=NS)
    out_flat = pl.kernel(
        _sc_kernel,
        out_type=jax.ShapeDtypeStruct((B * L * D,), jnp.float32),
        mesh=mesh,
        compiler_params=pltpu.CompilerParams(needs_layout_passes=False,
                                             use_tc_tiling_on_sc=False),
        scratch_types=[
            pltpu.VMEM((NGC, GCH), jnp.int32),
            pltpu.VMEM((NGC, GCH), jnp.int32),
            pltpu.VMEM((ROWS, D), jnp.float32),
            pltpu.VMEM((ROWS, D), jnp.float32),
            pltpu.VMEM((OUTF,), jnp.float32),
            pltpu.VMEM((OUTF,), jnp.float32),
            pltpu.VMEM((L * D,), jnp.float32),
            pltpu.VMEM((D,), jnp.float32),
            pltpu.VMEM((D,), jnp.float32),
            pltpu.SemaphoreType.DMA,
            pltpu.SemaphoreType.DMA,
            pltpu.SemaphoreType.DMA,
            pltpu.SemaphoreType.DMA,
            pltpu.SemaphoreType.DMA,
            pltpu.SemaphoreType.DMA,
        ],
    )(ids2, gene_table, pos_flat, ln_gamma, ln_beta)
    return out_flat.reshape(B, L, D)


# speedup vs baseline: 1.0009x; 1.0009x over previous
import jax
import jax.numpy as jnp
from jax import lax
from jax.experimental import pallas as pl
from jax.experimental.pallas import tpu as pltpu
from jax.experimental.pallas import tpu_sc as plsc

B = 4096
L = 200
D = 64
EPS = 1e-5

NC = 2
NS = 16
NW = NC * NS

BPW = B // NW
BPI = 2
NIT = BPW // BPI
ROWS = BPI * L
OUTF = ROWS * D
RB = 20
NRB = ROWS // RB
GCH = ((0, 104), (104, 96))

MAGIC = 0x5F3759DF


def _rsqrt(v):
    i = plsc.bitcast(v, jnp.int32)
    i = MAGIC - lax.shift_right_logical(i, 1)
    y = plsc.bitcast(i, jnp.float32)
    hv = 0.5 * v
    y = y * (1.5 - hv * y * y)
    y = y * (1.5 - hv * y * y)
    y = y * (1.5 - hv * y * y)
    return y


def _sc_kernel(ids_hbm, table_hbm, pos_hbm, gam_hbm, bet_hbm, out_hbm,
               idxA, idxB, inA, inB, outA, outB, pos_v, gam_v, bet_v,
               gsemA, gsemB, osemA, osemB, isemA, isemB):
    cid = lax.axis_index("c")
    sid = lax.axis_index("s")
    wid = cid * NS + sid

    idx = [idxA, idxB]
    inb = [inA, inB]
    outb = [outA, outB]
    gsem = [gsemA, gsemB]
    osem = [osemA, osemB]
    isem = [isemA, isemB]

    pltpu.sync_copy(pos_hbm.at[pl.ds(0, L * D)], pos_v)
    pltpu.sync_copy(gam_hbm, gam_v)
    pltpu.sync_copy(bet_hbm, bet_v)

    g = [gam_v[pl.ds(16 * k, 16)] for k in range(4)]
    bt = [bet_v[pl.ds(16 * k, 16)] for k in range(4)]

    def fire_ids(t, q):
        b0 = wid * BPW + t * BPI
        pltpu.async_copy(ids_hbm.at[pl.ds(b0, BPI), :], idx[q], isem[q])

    def wait_ids(q):
        pltpu.make_async_copy(ids_hbm.at[pl.ds(0, BPI), :], idx[q],
                              isem[q]).wait()

    def fire_gather(q):
        for j in range(BPI):
            for off, n in GCH:
                pltpu.async_copy(
                    table_hbm.at[idx[q].at[j, pl.ds(off, n)]],
                    inb[q].at[pl.ds(j * L + off, n), :], gsem[q])

    def wait_gather(q):
        pltpu.make_async_copy(table_hbm.at[pl.ds(0, ROWS), :], inb[q],
                              gsem[q]).wait()

    def fire_out(t, q):
        dst = (wid * NIT + t) * OUTF
        pltpu.async_copy(outb[q], out_hbm.at[pl.ds(dst, OUTF)], osem[q])

    def wait_out(q):
        pltpu.make_async_copy(out_hbm.at[pl.ds(0, OUTF)], outb[q],
                              osem[q]).wait()

    fire_ids(0, 0)
    wait_ids(0)
    fire_ids(1, 1)
    fire_gather(0)

    @pl.loop(0, NIT, step=2)
    def _iter2(t0):
        for p in (0, 1):
            t = t0 + p
            q = 1 - p

            @pl.when(t < NIT - 1)
            def _prefetch():
                wait_ids(q)
                fire_gather(q)

            wait_gather(p)

            @pl.when(t < NIT - 2)
            def _nextids():
                fire_ids(t + 2, p)

            @pl.when(t >= 2)
            def _drainout():
                wait_out(p)

            src = inb[p]
            dst = outb[p]

            @pl.loop(0, NRB)
            def _blk(blk):
                r0 = blk * RB
                l0 = lax.rem(r0, L)
                for i in range(RB):
                    row = r0 + i
                    po = (l0 + i) * D
                    x = [src[row, pl.ds(16 * k, 16)]
                         + pos_v[pl.ds(po + 16 * k, 16)]
                         for k in range(4)]
                    tot = (x[0] + x[1]) + (x[2] + x[3])
                    qq = ((x[0] * x[0] + x[1] * x[1])
                          + (x[2] * x[2] + x[3] * x[3]))
                    sv = jnp.full((16,), jnp.sum(tot))
                    qv = jnp.full((16,), jnp.sum(qq))
                    mean = sv * (1.0 / D)
                    var = qv * (1.0 / D) - mean * mean
                    rstd = _rsqrt(var + EPS)
                    base = row * D
                    for k in range(4):
                        y = (x[k] - mean) * (rstd * g[k]) + bt[k]
                        dst[pl.ds(base + 16 * k, 16)] = y

            fire_out(t, p)

    wait_out(0)
    wait_out(1)


@jax.jit
def kernel(input_ids_BL, gene_table, pos_table, ln_gamma, ln_beta):
    ids = input_ids_BL.astype(jnp.int32)
    pos_flat = pos_table.reshape(-1)

    mesh = plsc.VectorSubcoreMesh(core_axis_name="c", subcore_axis_name="s",
                                  num_cores=NC, num_subcores=NS)
    out_flat = pl.kernel(
        _sc_kernel,
        out_type=jax.ShapeDtypeStruct((B * L * D,), jnp.float32),
        mesh=mesh,
        compiler_params=pltpu.CompilerParams(needs_layout_passes=False,
                                             use_tc_tiling_on_sc=False),
        scratch_types=[
            pltpu.VMEM((BPI, L), jnp.int32),
            pltpu.VMEM((BPI, L), jnp.int32),
            pltpu.VMEM((ROWS, D), jnp.float32),
            pltpu.VMEM((ROWS, D), jnp.float32),
            pltpu.VMEM((OUTF,), jnp.float32),
            pltpu.VMEM((OUTF,), jnp.float32),
            pltpu.VMEM((L * D,), jnp.float32),
            pltpu.VMEM((D,), jnp.float32),
            pltpu.VMEM((D,), jnp.float32),
            pltpu.SemaphoreType.DMA,
            pltpu.SemaphoreType.DMA,
            pltpu.SemaphoreType.DMA,
            pltpu.SemaphoreType.DMA,
            pltpu.SemaphoreType.DMA,
            pltpu.SemaphoreType.DMA,
        ],
    )(ids, gene_table, pos_flat, ln_gamma, ln_beta)
    return out_flat.reshape(B, L, D)


# speedup vs baseline: 2.0043x; 2.0026x over previous
import jax
import jax.numpy as jnp
from jax import lax
from jax.experimental import pallas as pl
from jax.experimental.pallas import tpu as pltpu
from jax.experimental.pallas import tpu_sc as plsc

B = 4096
L = 200
D = 64
EPS = 1e-5

NC = 2
NS = 16
NW = NC * NS

BPW = B // NW
BPI = 2
NIT = BPW // BPI
ROWS = BPI * L
OUTF = ROWS * D
RB = 20
NRB = ROWS // RB
GCH = ((0, 104), (104, 96))

MAGIC = 0x5F3759DF


def _rsqrt(v):
    i = plsc.bitcast(v, jnp.int32)
    i = MAGIC - lax.shift_right_logical(i, 1)
    y = plsc.bitcast(i, jnp.float32)
    hv = 0.5 * v
    y = y * (1.5 - hv * y * y)
    y = y * (1.5 - hv * y * y)
    y = y * (1.5 - hv * y * y)
    return y


def _sc_kernel(ids_hbm, table_hbm, pos_hbm, gam_hbm, bet_hbm, out_hbm,
               idxA, idxB, inA, inB, outA, outB, pos_v, gam_v, bet_v,
               gsemA, gsemB, osemA, osemB, isemA, isemB):
    cid = lax.axis_index("c")
    sid = lax.axis_index("s")
    wid = cid * NS + sid

    idx = [idxA, idxB]
    inb = [inA, inB]
    outb = [outA, outB]
    gsem = [gsemA, gsemB]
    osem = [osemA, osemB]
    isem = [isemA, isemB]

    pltpu.sync_copy(pos_hbm.at[pl.ds(0, L * D)], pos_v)
    pltpu.sync_copy(gam_hbm, gam_v)
    pltpu.sync_copy(bet_hbm, bet_v)

    g = [gam_v[pl.ds(16 * k, 16)] for k in range(4)]
    bt = [bet_v[pl.ds(16 * k, 16)] for k in range(4)]

    def fire_ids(t, q):
        b0 = wid * BPW + t * BPI
        pltpu.async_copy(ids_hbm.at[pl.ds(b0, BPI), :], idx[q], isem[q])

    def wait_ids(q):
        pltpu.make_async_copy(ids_hbm.at[pl.ds(0, BPI), :], idx[q],
                              isem[q]).wait()

    def fire_gather(q):
        for j in range(BPI):
            for off, n in GCH:
                pltpu.async_copy(
                    table_hbm.at[idx[q].at[j, pl.ds(off, n)]],
                    inb[q].at[pl.ds(j * L + off, n), :], gsem[q])

    def wait_gather(q):
        pltpu.make_async_copy(table_hbm.at[pl.ds(0, ROWS), :], inb[q],
                              gsem[q]).wait()

    def fire_out(t, q):
        dst = (wid * NIT + t) * OUTF
        pltpu.async_copy(outb[q], out_hbm.at[pl.ds(dst, OUTF)], osem[q])

    def wait_out(q):
        pltpu.make_async_copy(out_hbm.at[pl.ds(0, OUTF)], outb[q],
                              osem[q]).wait()

    fire_ids(0, 0)
    wait_ids(0)
    fire_ids(1, 1)
    fire_gather(0)

    @pl.loop(0, NIT, step=2)
    def _iter2(t0):
        for p in (0, 1):
            t = t0 + p
            q = 1 - p

            @pl.when(t < NIT - 1)
            def _prefetch():
                wait_ids(q)
                fire_gather(q)

            wait_gather(p)

            @pl.when(t < NIT - 2)
            def _nextids():
                fire_ids(t + 2, p)

            @pl.when(t >= 2)
            def _drainout():
                wait_out(p)

            src = inb[p]
            dst = outb[p]

            @plsc.parallel_loop(0, ROWS, unroll=4)
            def _row(row):
                po = lax.rem(row, L) * D
                x = [src[row, pl.ds(16 * k, 16)]
                     + pos_v[pl.ds(po + 16 * k, 16)]
                     for k in range(4)]
                tot = (x[0] + x[1]) + (x[2] + x[3])
                qq = ((x[0] * x[0] + x[1] * x[1])
                      + (x[2] * x[2] + x[3] * x[3]))
                sv = jnp.full((16,), jnp.sum(tot))
                qv = jnp.full((16,), jnp.sum(qq))
                mean = sv * (1.0 / D)
                var = qv * (1.0 / D) - mean * mean
                rstd = _rsqrt(var + EPS)
                base = row * D
                for k in range(4):
                    y = (x[k] - mean) * (rstd * g[k]) + bt[k]
                    dst[pl.ds(base + 16 * k, 16)] = y

            fire_out(t, p)

    wait_out(0)
    wait_out(1)


@jax.jit
def kernel(input_ids_BL, gene_table, pos_table, ln_gamma, ln_beta):
    ids = input_ids_BL.astype(jnp.int32)
    pos_flat = pos_table.reshape(-1)

    mesh = plsc.VectorSubcoreMesh(core_axis_name="c", subcore_axis_name="s",
                                  num_cores=NC, num_subcores=NS)
    out_flat = pl.kernel(
        _sc_kernel,
        out_type=jax.ShapeDtypeStruct((B * L * D,), jnp.float32),
        mesh=mesh,
        compiler_params=pltpu.CompilerParams(needs_layout_passes=False,
                                             use_tc_tiling_on_sc=False),
        scratch_types=[
            pltpu.VMEM((BPI, L), jnp.int32),
            pltpu.VMEM((BPI, L), jnp.int32),
            pltpu.VMEM((ROWS, D), jnp.float32),
            pltpu.VMEM((ROWS, D), jnp.float32),
            pltpu.VMEM((OUTF,), jnp.float32),
            pltpu.VMEM((OUTF,), jnp.float32),
            pltpu.VMEM((L * D,), jnp.float32),
            pltpu.VMEM((D,), jnp.float32),
            pltpu.VMEM((D,), jnp.float32),
            pltpu.SemaphoreType.DMA,
            pltpu.SemaphoreType.DMA,
            pltpu.SemaphoreType.DMA,
            pltpu.SemaphoreType.DMA,
            pltpu.SemaphoreType.DMA,
            pltpu.SemaphoreType.DMA,
        ],
    )(ids, gene_table, pos_flat, ln_gamma, ln_beta)
    return out_flat.reshape(B, L, D)
